# baseline (speedup 1.0000x reference)
import numpy as np

import jax
import jax.numpy as jnp
from jax.experimental import pallas as pl
from jax.experimental.pallas import tpu as pltpu

_NB = 128
_F1 = 4096
_F2 = 2048


def _onehot_updown(n_in, n_out):
    m = np.zeros((n_in, n_out, 4), np.float32)
    for i in range(n_out):
        for r in range(4):
            h = 2 * i + r
            if h < n_in:
                m[h, i, r] = 1.0
    return m


_IH1 = _onehot_updown(28, 13)
_IH2 = _onehot_updown(13, 5)


def _quad_taps(w_hw):
    parts = []
    for u in (0, 1):
        for v in (0, 1):
            pad = [(0, 0)] * (w_hw.ndim - 2) + [(u, 1 - u), (v, 1 - v)]
            parts.append(jnp.pad(w_hw, pad))
    q = jnp.stack(parts, axis=-1)
    nd = q.ndim
    return jnp.moveaxis(q, (nd - 3, nd - 2), (0, 1))


def _lenet_body(x_ref, w1_ref, b1_ref, w2_ref, b2_ref,
                f1_ref, g1_ref, f2_ref, g2_ref, f3_ref, g3_ref, o_ref):
    f32 = jnp.float32
    bf16 = jnp.bfloat16
    xb = x_ref[...].astype(bf16)
    o1 = jnp.dot(xb, w1_ref[...], preferred_element_type=f32)
    m1 = jnp.maximum(jnp.maximum(o1[:, 0:1024], o1[:, 1024:2048]),
                     jnp.maximum(o1[:, 2048:3072], o1[:, 3072:4096]))
    m1 = jnp.maximum(m1 + b1_ref[...], 0.0).astype(bf16)
    o2 = jnp.dot(m1, w2_ref[...], preferred_element_type=f32)
    m2 = jnp.maximum(jnp.maximum(o2[:, 0:512], o2[:, 512:1024]),
                     jnp.maximum(o2[:, 1024:1536], o2[:, 1536:2048]))
    m2 = jnp.maximum(m2 + b2_ref[...], 0.0).astype(bf16)
    h1 = jnp.maximum(jnp.dot(m2, f1_ref[...], preferred_element_type=f32)
                     + g1_ref[...], 0.0).astype(bf16)
    h2 = jnp.maximum(jnp.dot(h1, f2_ref[...], preferred_element_type=f32)
                     + g2_ref[...], 0.0).astype(bf16)
    o_ref[...] = jnp.dot(h2, f3_ref[...], preferred_element_type=f32) + g3_ref[...]


def kernel(x, w1, b1, w2, b2, wf1, bf1, wf2, bf2, wf3, bf3):
    f32 = jnp.float32
    bf16 = jnp.bfloat16
    B = x.shape[0]
    nb = _NB if B % _NB == 0 else B
    x2d = x.reshape(B, 784)

    w1p = _quad_taps(w1[:, 0])
    w1full = 0.0
    for a in range(4):
        ah = _IH1[:, :, a][:, None, None, :, None, None]
        for b in range(4):
            bw = _IH1[:, :, b][None, :, None, None, :, None]
            tap = jnp.transpose(w1p[a, b])[None, None, :, None, None, :]
            w1full = w1full + (ah * bw) * tap
    w1d = jnp.pad(w1full.astype(bf16).reshape(784, 4, 1014),
                  ((0, 0), (0, 0), (0, 10))).reshape(784, _F1)
    b1d = jnp.pad(jnp.broadcast_to(b1, (169, 6)).reshape(1, 1014),
                  ((0, 0), (0, 10)))

    w2p = _quad_taps(w2)
    w2full = 0.0
    for r in range(4):
        ih = _IH2[:, :, r][:, None, None, None, :, None, None]
        for s in range(4):
            jw = _IH2[:, :, s][None, :, None, None, None, :, None]
            tap = jnp.transpose(w2p[r, s], (1, 2, 0))[None, None, :, :, None, None, :]
            w2full = w2full + (ih * jw) * tap
    w2d = jnp.pad(w2full.astype(bf16).reshape(1014, 4, 400),
                  ((0, 10), (0, 0), (0, 112))).reshape(1024, _F2)
    b2d = jnp.pad(jnp.broadcast_to(b2, (25, 16)).reshape(1, 400),
                  ((0, 0), (0, 112)))

    wf1p = jnp.pad(wf1.reshape(16, 5, 5, 120).transpose(1, 2, 0, 3).reshape(400, 120),
                   ((0, 112), (0, 8))).astype(bf16)
    bf1p = jnp.pad(bf1, (0, 8)).reshape(1, 128)
    wf2p = jnp.pad(wf2, ((0, 8), (0, 44))).astype(bf16)
    bf2p = jnp.pad(bf2, (0, 44)).reshape(1, 128)
    wf3p = jnp.pad(wf3, ((0, 44), (0, 118))).astype(bf16)
    bf3p = jnp.pad(bf3, (0, 118)).reshape(1, 128)

    const = lambda: (lambda b: (0, 0))
    out = pl.pallas_call(
        _lenet_body,
        out_shape=jax.ShapeDtypeStruct((B, 128), f32),
        grid=(B // nb,),
        in_specs=[
            pl.BlockSpec((nb, 784), lambda b: (b, 0)),
            pl.BlockSpec((784, _F1), const()),
            pl.BlockSpec((1, 1024), const()),
            pl.BlockSpec((1024, _F2), const()),
            pl.BlockSpec((1, 512), const()),
            pl.BlockSpec((512, 128), const()),
            pl.BlockSpec((1, 128), const()),
            pl.BlockSpec((128, 128), const()),
            pl.BlockSpec((1, 128), const()),
            pl.BlockSpec((128, 128), const()),
            pl.BlockSpec((1, 128), const()),
        ],
        out_specs=pl.BlockSpec((nb, 128), lambda b: (b, 0)),
        compiler_params=pltpu.CompilerParams(
            dimension_semantics=("parallel",),
            vmem_limit_bytes=100 * 1024 * 1024,
        ),
    )(x2d, w1d, b1d, w2d, b2d, wf1p, bf1p, wf2p, bf2p, wf3p, bf3p)
    return out[:, :10]


# speedup vs baseline: 3.6543x; 3.6543x over previous
import numpy as np

import jax
import jax.numpy as jnp
from jax.experimental import pallas as pl
from jax.experimental.pallas import tpu as pltpu

_NB = 128
_F1 = 4096
_F2 = 2048


def _onehot_updown(n_in, n_out):
    m = np.zeros((n_in, n_out, 4), np.float32)
    for i in range(n_out):
        for r in range(4):
            h = 2 * i + r
            if h < n_in:
                m[h, i, r] = 1.0
    return m


_IH1 = _onehot_updown(28, 13)
_IH2 = _onehot_updown(13, 5)

_KC1 = np.einsum("hia,wjb->hwijab", _IH1, _IH1).reshape(784 * 169, 16)
_KC2M = np.einsum("iAr,jBs->ijABrs", _IH2, _IH2).reshape(169 * 25, 16)


def _quad_taps(w_hw):
    parts = []
    for u in (0, 1):
        for v in (0, 1):
            pad = [(0, 0)] * (w_hw.ndim - 2) + [(u, 1 - u), (v, 1 - v)]
            parts.append(jnp.pad(w_hw, pad))
    q = jnp.stack(parts, axis=-1)
    nd = q.ndim
    return jnp.moveaxis(q, (nd - 3, nd - 2), (0, 1))


def _lenet_body(x_ref, w1_ref, b1_ref, w2_ref, b2_ref,
                f1_ref, g1_ref, f2_ref, g2_ref, f3_ref, g3_ref, o_ref):
    f32 = jnp.float32
    bf16 = jnp.bfloat16
    o1 = jnp.dot(x_ref[...], w1_ref[...], preferred_element_type=f32)
    m1 = jnp.maximum(jnp.maximum(o1[:, 0:1024], o1[:, 1024:2048]),
                     jnp.maximum(o1[:, 2048:3072], o1[:, 3072:4096]))
    m1 = jnp.maximum(m1 + b1_ref[...], 0.0).astype(bf16)
    o2 = jnp.dot(m1, w2_ref[...], preferred_element_type=f32)
    m2 = jnp.maximum(jnp.maximum(o2[:, 0:512], o2[:, 512:1024]),
                     jnp.maximum(o2[:, 1024:1536], o2[:, 1536:2048]))
    m2 = jnp.maximum(m2 + b2_ref[...], 0.0).astype(bf16)
    h1 = jnp.maximum(jnp.dot(m2, f1_ref[...], preferred_element_type=f32)
                     + g1_ref[...], 0.0).astype(bf16)
    h2 = jnp.maximum(jnp.dot(h1, f2_ref[...], preferred_element_type=f32)
                     + g2_ref[...], 0.0).astype(bf16)
    o_ref[...] = jnp.dot(h2, f3_ref[...], preferred_element_type=f32) + g3_ref[...]


def kernel(x, w1, b1, w2, b2, wf1, bf1, wf2, bf2, wf3, bf3):
    f32 = jnp.float32
    bf16 = jnp.bfloat16
    B = x.shape[0]
    nb = _NB if B % _NB == 0 else B
    x2d = x.reshape(B, 784).astype(bf16)

    w1p = _quad_taps(w1[:, 0]).astype(bf16)
    kc1 = jnp.asarray(_KC1, bf16)
    blocks1 = []
    for q in range(4):
        z = jnp.dot(kc1, w1p[:, :, :, q].reshape(16, 6),
                    preferred_element_type=f32)
        blocks1.append(jnp.pad(z.astype(bf16).reshape(784, 1014),
                               ((0, 0), (0, 10))))
    w1d = jnp.concatenate(blocks1, axis=1)
    b1d = jnp.pad(jnp.broadcast_to(b1, (169, 6)).reshape(1, 1014),
                  ((0, 0), (0, 10)))

    w2p = _quad_taps(w2).astype(bf16)
    kc2 = jnp.asarray(_KC2M, bf16)
    blocks2 = []
    for q in range(4):
        tap = jnp.transpose(w2p[:, :, :, :, q], (0, 1, 3, 2)).reshape(16, 96)
        z = jnp.dot(kc2, tap, preferred_element_type=f32)
        z = jnp.transpose(z.astype(bf16).reshape(169, 25, 6, 16), (0, 2, 1, 3))
        blocks2.append(jnp.pad(z.reshape(1014, 400), ((0, 10), (0, 112))))
    w2d = jnp.concatenate(blocks2, axis=1)
    b2d = jnp.pad(jnp.broadcast_to(b2, (25, 16)).reshape(1, 400),
                  ((0, 0), (0, 112)))

    wf1p = jnp.pad(wf1.reshape(16, 5, 5, 120).transpose(1, 2, 0, 3).reshape(400, 120),
                   ((0, 112), (0, 8))).astype(bf16)
    bf1p = jnp.pad(bf1, (0, 8)).reshape(1, 128)
    wf2p = jnp.pad(wf2, ((0, 8), (0, 44))).astype(bf16)
    bf2p = jnp.pad(bf2, (0, 44)).reshape(1, 128)
    wf3p = jnp.pad(wf3, ((0, 44), (0, 118))).astype(bf16)
    bf3p = jnp.pad(bf3, (0, 118)).reshape(1, 128)

    const = lambda: (lambda b: (0, 0))
    out = pl.pallas_call(
        _lenet_body,
        out_shape=jax.ShapeDtypeStruct((B, 128), f32),
        grid=(B // nb,),
        in_specs=[
            pl.BlockSpec((nb, 784), lambda b: (b, 0)),
            pl.BlockSpec((784, _F1), const()),
            pl.BlockSpec((1, 1024), const()),
            pl.BlockSpec((1024, _F2), const()),
            pl.BlockSpec((1, 512), const()),
            pl.BlockSpec((512, 128), const()),
            pl.BlockSpec((1, 128), const()),
            pl.BlockSpec((128, 128), const()),
            pl.BlockSpec((1, 128), const()),
            pl.BlockSpec((128, 128), const()),
            pl.BlockSpec((1, 128), const()),
        ],
        out_specs=pl.BlockSpec((nb, 128), lambda b: (b, 0)),
        compiler_params=pltpu.CompilerParams(
            dimension_semantics=("parallel",),
            vmem_limit_bytes=100 * 1024 * 1024,
        ),
    )(x2d, w1d, b1d, w2d, b2d, wf1p, bf1p, wf2p, bf2p, wf3p, bf3p)
    return out[:, :10]


# speedup vs baseline: 4.1113x; 1.1250x over previous
import numpy as np

import jax
import jax.numpy as jnp
from jax.experimental import pallas as pl
from jax.experimental.pallas import tpu as pltpu

_NB = 256
_F1 = 4096
_F2 = 2048


def _onehot_updown(n_in, n_out):
    m = np.zeros((n_in, n_out, 4), np.float32)
    for i in range(n_out):
        for r in range(4):
            h = 2 * i + r
            if h < n_in:
                m[h, i, r] = 1.0
    return m


_IH1 = _onehot_updown(28, 13)
_IH2 = _onehot_updown(13, 5)

_KC1 = np.einsum("hia,wjb->hwijab", _IH1, _IH1).reshape(784 * 169, 16)
_KC2M = np.einsum("iAr,jBs->ijABrs", _IH2, _IH2).reshape(169 * 25, 16)


def _quad_taps(w_hw):
    parts = []
    for u in (0, 1):
        for v in (0, 1):
            pad = [(0, 0)] * (w_hw.ndim - 2) + [(u, 1 - u), (v, 1 - v)]
            parts.append(jnp.pad(w_hw, pad))
    q = jnp.stack(parts, axis=-1)
    nd = q.ndim
    return jnp.moveaxis(q, (nd - 3, nd - 2), (0, 1))


def _lenet_body(x_ref, w1_ref, b1_ref, w2_ref, b2_ref,
                f1_ref, g1_ref, f2_ref, g2_ref, f3_ref, g3_ref, o_ref):
    f32 = jnp.float32
    bf16 = jnp.bfloat16
    o1 = jnp.dot(x_ref[...].astype(bf16), w1_ref[...],
                 preferred_element_type=f32)
    m1 = jnp.maximum(jnp.maximum(o1[:, 0:1024], o1[:, 1024:2048]),
                     jnp.maximum(o1[:, 2048:3072], o1[:, 3072:4096]))
    m1 = jnp.maximum(m1 + b1_ref[...], 0.0).astype(bf16)
    o2 = jnp.dot(m1, w2_ref[...], preferred_element_type=f32)
    m2 = jnp.maximum(jnp.maximum(o2[:, 0:512], o2[:, 512:1024]),
                     jnp.maximum(o2[:, 1024:1536], o2[:, 1536:2048]))
    m2 = jnp.maximum(m2 + b2_ref[...], 0.0).astype(bf16)
    h1 = jnp.maximum(jnp.dot(m2, f1_ref[...], preferred_element_type=f32)
                     + g1_ref[...], 0.0).astype(bf16)
    h2 = jnp.maximum(jnp.dot(h1, f2_ref[...], preferred_element_type=f32)
                     + g2_ref[...], 0.0).astype(bf16)
    o_ref[...] = jnp.dot(h2, f3_ref[...], preferred_element_type=f32) + g3_ref[...]


def kernel(x, w1, b1, w2, b2, wf1, bf1, wf2, bf2, wf3, bf3):
    f32 = jnp.float32
    bf16 = jnp.bfloat16
    B = x.shape[0]
    nb = _NB if B % _NB == 0 else B
    x2d = x.reshape(B, 784)

    ih1 = _IH1.astype(jnp.bfloat16)
    w1p = _quad_taps(w1[:, 0]).astype(bf16)
    t1 = jnp.einsum("hir,rscq->hiscq", ih1, w1p,
                    preferred_element_type=bf16)
    w1full = jnp.einsum("wjs,hiscq->hwqijc", ih1, t1,
                        preferred_element_type=bf16)
    w1d = jnp.pad(w1full.reshape(784, 4, 1014),
                  ((0, 0), (0, 0), (0, 10))).reshape(784, _F1)
    b1d = jnp.pad(jnp.broadcast_to(b1, (169, 6)).reshape(1, 1014),
                  ((0, 0), (0, 10)))

    ih2 = _IH2.astype(jnp.bfloat16)
    w2p = _quad_taps(w2).astype(bf16)
    t2 = jnp.einsum("iAr,rsbaq->iAsbaq", ih2, w2p,
                    preferred_element_type=bf16)
    w2full = jnp.einsum("jBs,iAsbaq->ijaqABb", ih2, t2,
                        preferred_element_type=bf16)
    w2d = jnp.pad(w2full.reshape(1014, 4, 400),
                  ((0, 10), (0, 0), (0, 112))).reshape(1024, _F2)
    b2d = jnp.pad(jnp.broadcast_to(b2, (25, 16)).reshape(1, 400),
                  ((0, 0), (0, 112)))

    wf1p = jnp.pad(wf1.reshape(16, 5, 5, 120).transpose(1, 2, 0, 3).reshape(400, 120),
                   ((0, 112), (0, 8))).astype(bf16)
    bf1p = jnp.pad(bf1, (0, 8)).reshape(1, 128)
    wf2p = jnp.pad(wf2, ((0, 8), (0, 44))).astype(bf16)
    bf2p = jnp.pad(bf2, (0, 44)).reshape(1, 128)
    wf3p = jnp.pad(wf3, ((0, 44), (0, 118))).astype(bf16)
    bf3p = jnp.pad(bf3, (0, 118)).reshape(1, 128)

    const = lambda: (lambda b: (0, 0))
    out = pl.pallas_call(
        _lenet_body,
        out_shape=jax.ShapeDtypeStruct((B, 128), f32),
        grid=(B // nb,),
        in_specs=[
            pl.BlockSpec((nb, 784), lambda b: (b, 0)),
            pl.BlockSpec((784, _F1), const()),
            pl.BlockSpec((1, 1024), const()),
            pl.BlockSpec((1024, _F2), const()),
            pl.BlockSpec((1, 512), const()),
            pl.BlockSpec((512, 128), const()),
            pl.BlockSpec((1, 128), const()),
            pl.BlockSpec((128, 128), const()),
            pl.BlockSpec((1, 128), const()),
            pl.BlockSpec((128, 128), const()),
            pl.BlockSpec((1, 128), const()),
        ],
        out_specs=pl.BlockSpec((nb, 128), lambda b: (b, 0)),
        compiler_params=pltpu.CompilerParams(
            dimension_semantics=("parallel",),
            vmem_limit_bytes=100 * 1024 * 1024,
        ),
    )(x2d, w1d, b1d, w2d, b2d, wf1p, bf1p, wf2p, bf2p, wf3p, bf3p)
    return out[:, :10]
